# baseline (device time: 1642975 ns/iter reference)
import jax
import jax.numpy as jnp
from jax import lax
from jax.experimental import pallas as pl
from jax.experimental.pallas import tpu as pltpu

N_DEV = 32


def _gelu(y):
    c = 0.7978845608028654
    return 0.5 * y * (1.0 + jnp.tanh(c * (y + 0.044715 * y * y * y)))


def kernel(x, w_mat):
    m, _ = x.shape
    _, n = w_mat.shape
    m_per = m // N_DEV

    def body(x_ref, w_ref, out_ref, buf, tmp, send_sems, recv_sems, credit_sems):
        my = lax.axis_index("i")
        left = lax.rem(my + (N_DEV - 1), N_DEV)
        right = lax.rem(my + 1, N_DEV)

        barrier = pltpu.get_barrier_semaphore()
        for nbr in (left, right):
            pl.semaphore_signal(
                barrier, inc=1,
                device_id=(nbr,), device_id_type=pl.DeviceIdType.MESH,
            )
        pl.semaphore_wait(barrier, 2)

        def partial(c):
            blk = x_ref[pl.ds(c * m_per, m_per), :]
            return jnp.dot(blk, w_ref[:, :], preferred_element_type=jnp.float32)

        buf[0, :, :] = partial(lax.rem(my + (N_DEV - 1), N_DEV))

        for s in range(N_DEV - 1):
            send_slot = s % 2
            recv_slot = (s + 1) % 2
            rdma = pltpu.make_async_remote_copy(
                src_ref=buf.at[send_slot],
                dst_ref=buf.at[recv_slot],
                send_sem=send_sems.at[send_slot],
                recv_sem=recv_sems.at[recv_slot],
                device_id=(right,),
                device_id_type=pl.DeviceIdType.MESH,
            )
            if s >= 1:
                pl.semaphore_wait(credit_sems.at[recv_slot], 1)
            rdma.start()
            tmp[:, :] = partial(lax.rem(my + ((N_DEV - 2 - s) % N_DEV), N_DEV))
            rdma.wait_recv()
            if s < N_DEV - 2:
                buf[recv_slot, :, :] = buf[recv_slot, :, :] + tmp[:, :]
            else:
                out_ref[:, :] = _gelu(buf[recv_slot, :, :] + tmp[:, :])
            rdma.wait_send()
            if s < N_DEV - 2:
                pl.semaphore_signal(
                    credit_sems.at[send_slot], inc=1,
                    device_id=(left,), device_id_type=pl.DeviceIdType.MESH,
                )

    return pl.pallas_call(
        body,
        out_shape=jax.ShapeDtypeStruct((m_per, n), jnp.float32),
        in_specs=[
            pl.BlockSpec(memory_space=pltpu.VMEM),
            pl.BlockSpec(memory_space=pltpu.VMEM),
        ],
        out_specs=pl.BlockSpec(memory_space=pltpu.VMEM),
        scratch_shapes=[
            pltpu.VMEM((2, m_per, n), jnp.float32),
            pltpu.VMEM((m_per, n), jnp.float32),
            pltpu.SemaphoreType.DMA((2,)),
            pltpu.SemaphoreType.DMA((2,)),
            pltpu.SemaphoreType.REGULAR((2,)),
        ],
        compiler_params=pltpu.CompilerParams(collective_id=0),
    )(x, w_mat)


# device time: 801328 ns/iter; 2.0503x vs baseline; 2.0503x over previous
import jax
import jax.numpy as jnp
from jax import lax
from jax.experimental import pallas as pl
from jax.experimental.pallas import tpu as pltpu

N_DEV = 32


def _ring_tables():
    all_coords = sorted((x, y, z) for x in range(2) for y in range(4) for z in range(4))
    order = []
    for z in sorted({c[2] for c in all_coords}):
        plane = sorted(c for c in all_coords if c[2] == z)
        for yi, y in enumerate(sorted({c[1] for c in plane})):
            row = sorted((c for c in plane if c[1] == y), reverse=bool(yi % 2))
            order.extend(row)
    logical_of_coords = {c: i for i, c in enumerate(order)}

    yz = [(0, 0), (0, 1), (0, 2), (0, 3), (1, 3), (1, 2), (1, 1), (2, 1),
          (2, 2), (2, 3), (3, 3), (3, 2), (3, 1), (3, 0), (2, 0), (1, 0)]
    cycle = [(0, y, z) for (y, z) in yz] + [(1, y, z) for (y, z) in reversed(yz)]
    log_of_rr = [logical_of_coords[c] for c in cycle]
    rr_of_log = [0] * N_DEV
    for r, l in enumerate(log_of_rr):
        rr_of_log[l] = r
    return log_of_rr, rr_of_log


def _gelu(y):
    c = 0.7978845608028654
    return 0.5 * y * (1.0 + jnp.tanh(c * (y + 0.044715 * y * y * y)))


def kernel(x, w_mat):
    m, _ = x.shape
    _, n = w_mat.shape
    m_per = m // N_DEV
    nh = n // 2

    log_of_rr, rr_of_log = _ring_tables()
    log_tab = jnp.asarray(log_of_rr, jnp.int32)
    rr_tab = jnp.asarray(rr_of_log, jnp.int32)

    def body(log_ref, rr_ref, x_ref, w_ref, out_ref,
             buf_r, buf_l, tmp_r, tmp_l,
             ssem_r, rsem_r, ssem_l, rsem_l, cred_r, cred_l):
        my = lax.axis_index("i")
        rr = rr_ref[my]
        right = log_ref[lax.rem(rr + 1, N_DEV)]
        left = log_ref[lax.rem(rr + (N_DEV - 1), N_DEV)]

        barrier = pltpu.get_barrier_semaphore()
        for nbr in (left, right):
            pl.semaphore_signal(
                barrier, inc=1,
                device_id=(nbr,), device_id_type=pl.DeviceIdType.MESH,
            )
        pl.semaphore_wait(barrier, 2)

        def partial_r(c):
            blk = x_ref[pl.ds(c * m_per, m_per), :]
            return jnp.dot(blk, w_ref[:, :nh], preferred_element_type=jnp.float32)

        def partial_l(c):
            blk = x_ref[pl.ds(c * m_per, m_per), :]
            return jnp.dot(blk, w_ref[:, nh:], preferred_element_type=jnp.float32)

        buf_r[0, :, :] = partial_r(left)
        buf_l[0, :, :] = partial_l(right)

        for s in range(N_DEV - 1):
            ss = s % 2
            rs = (s + 1) % 2
            rdma_r = pltpu.make_async_remote_copy(
                src_ref=buf_r.at[ss], dst_ref=buf_r.at[rs],
                send_sem=ssem_r.at[ss], recv_sem=rsem_r.at[rs],
                device_id=(right,), device_id_type=pl.DeviceIdType.MESH,
            )
            rdma_l = pltpu.make_async_remote_copy(
                src_ref=buf_l.at[ss], dst_ref=buf_l.at[rs],
                send_sem=ssem_l.at[ss], recv_sem=rsem_l.at[rs],
                device_id=(left,), device_id_type=pl.DeviceIdType.MESH,
            )
            if s >= 1:
                pl.semaphore_wait(cred_r.at[rs], 1)
                pl.semaphore_wait(cred_l.at[rs], 1)
            rdma_r.start()
            rdma_l.start()
            c_r = log_ref[lax.rem(rr + ((N_DEV - 2 - s) % N_DEV), N_DEV)]
            c_l = log_ref[lax.rem(rr + ((2 + s) % N_DEV), N_DEV)]
            tmp_r[:, :] = partial_r(c_r)
            tmp_l[:, :] = partial_l(c_l)
            rdma_r.wait_recv()
            rdma_l.wait_recv()
            if s < N_DEV - 2:
                buf_r[rs, :, :] = buf_r[rs, :, :] + tmp_r[:, :]
                buf_l[rs, :, :] = buf_l[rs, :, :] + tmp_l[:, :]
            else:
                out_ref[:, :nh] = _gelu(buf_r[rs, :, :] + tmp_r[:, :])
                out_ref[:, nh:] = _gelu(buf_l[rs, :, :] + tmp_l[:, :])
            rdma_r.wait_send()
            rdma_l.wait_send()
            if s < N_DEV - 2:
                pl.semaphore_signal(
                    cred_r.at[ss], inc=1,
                    device_id=(left,), device_id_type=pl.DeviceIdType.MESH,
                )
                pl.semaphore_signal(
                    cred_l.at[ss], inc=1,
                    device_id=(right,), device_id_type=pl.DeviceIdType.MESH,
                )

    return pl.pallas_call(
        body,
        out_shape=jax.ShapeDtypeStruct((m_per, n), jnp.float32),
        in_specs=[
            pl.BlockSpec(memory_space=pltpu.SMEM),
            pl.BlockSpec(memory_space=pltpu.SMEM),
            pl.BlockSpec(memory_space=pltpu.VMEM),
            pl.BlockSpec(memory_space=pltpu.VMEM),
        ],
        out_specs=pl.BlockSpec(memory_space=pltpu.VMEM),
        scratch_shapes=[
            pltpu.VMEM((2, m_per, nh), jnp.float32),
            pltpu.VMEM((2, m_per, nh), jnp.float32),
            pltpu.VMEM((m_per, nh), jnp.float32),
            pltpu.VMEM((m_per, nh), jnp.float32),
            pltpu.SemaphoreType.DMA((2,)),
            pltpu.SemaphoreType.DMA((2,)),
            pltpu.SemaphoreType.DMA((2,)),
            pltpu.SemaphoreType.DMA((2,)),
            pltpu.SemaphoreType.REGULAR((2,)),
            pltpu.SemaphoreType.REGULAR((2,)),
        ],
        compiler_params=pltpu.CompilerParams(collective_id=0),
    )(log_tab, rr_tab, x, w_mat)
